# Initial kernel scaffold; baseline (speedup 1.0000x reference)
import jax
import jax.numpy as jnp
from jax.experimental import pallas as pl


def kernel(x):
    raise NotImplementedError("write your pallas kernel here")


# speedup vs baseline: 1.9691x; 1.9691x over previous
import math
from functools import lru_cache, partial

import numpy as np
import jax
import jax.numpy as jnp
from jax.experimental import pallas as pl
from jax.experimental.pallas import tpu as pltpu

_VMEM_LIMIT_BYTES = 48 * 1024 * 1024


def _nearest_indices(in_dim: int, out_dim: int) -> np.ndarray:
    src = np.floor(np.arange(out_dim, dtype=np.float32) * np.float32(in_dim / out_dim))
    return np.clip(src.astype(np.int64), 0, in_dim - 1)


@lru_cache(maxsize=16)
def _sel_w_mat(w_in: int, w_out: int):
    idx = _nearest_indices(w_in, w_out)
    m = np.zeros((w_in, w_out), dtype=np.float32)
    m[idx, np.arange(w_out)] = 1.0
    return jnp.asarray(m)


def _upsample_kernel(sel_w_ref, x_ref, o_ref, *, sf_h):
    t = jnp.dot(x_ref[...], sel_w_ref[...], preferred_element_type=jnp.float32)
    r, w = t.shape
    o_ref[...] = jnp.broadcast_to(t[:, None, :], (r, sf_h, w)).reshape(r * sf_h, w)


def kernel(x):
    N, C, H_in, W_in = x.shape
    sf_h = sf_w = 2
    H_out, W_out = H_in * sf_h, W_in * sf_w

    orig_dtype = x.dtype
    if not jnp.issubdtype(x.dtype, jnp.floating):
        x = x.astype(jnp.float32)

    NC = N * C
    c_blk = 16
    while NC % c_blk:
        c_blk //= 2
    grid = NC // c_blk

    x2d = x.reshape(NC * H_in, W_in)
    sel_w = _sel_w_mat(W_in, W_out).astype(x.dtype)

    out2d = pl.pallas_call(
        partial(_upsample_kernel, sf_h=sf_h),
        out_shape=jax.ShapeDtypeStruct((NC * H_out, W_out), x.dtype),
        grid=(grid,),
        in_specs=[
            pl.BlockSpec((W_in, W_out), lambda i: (0, 0)),
            pl.BlockSpec((c_blk * H_in, W_in), lambda i: (i, 0)),
        ],
        out_specs=pl.BlockSpec((c_blk * H_out, W_out), lambda i: (i, 0)),
        compiler_params=pltpu.CompilerParams(
            dimension_semantics=("parallel",),
            vmem_limit_bytes=_VMEM_LIMIT_BYTES,
        ),
    )(sel_w, x2d)

    out = out2d.reshape(N, C, H_out, W_out)
    if out.dtype != orig_dtype:
        out = out.astype(orig_dtype)
    return out


# speedup vs baseline: 2.7316x; 1.3872x over previous
import math
from functools import lru_cache, partial

import numpy as np
import jax
import jax.numpy as jnp
from jax.experimental import pallas as pl
from jax.experimental.pallas import tpu as pltpu

_VMEM_LIMIT_BYTES = 48 * 1024 * 1024


def _nearest_indices(in_dim: int, out_dim: int) -> np.ndarray:
    src = np.floor(np.arange(out_dim, dtype=np.float32) * np.float32(in_dim / out_dim))
    return np.clip(src.astype(np.int64), 0, in_dim - 1)


@lru_cache(maxsize=16)
def _sel_w_mat(w_in: int, w_out: int):
    idx = _nearest_indices(w_in, w_out)
    m = np.zeros((w_in, w_out), dtype=np.float32)
    m[idx, np.arange(w_out)] = 1.0
    return jnp.asarray(m)


def _upsample_kernel(sel_w_ref, x_ref, o_ref, *, sf_h):
    t = jnp.dot(x_ref[...], sel_w_ref[...], preferred_element_type=jnp.float32)
    r, w = t.shape
    o_ref[...] = jnp.broadcast_to(t[:, None, :], (r, sf_h, w)).reshape(r * sf_h, w)


def kernel(x):
    N, C, H_in, W_in = x.shape
    sf_h = sf_w = 2
    H_out, W_out = H_in * sf_h, W_in * sf_w

    orig_dtype = x.dtype
    if not jnp.issubdtype(x.dtype, jnp.floating):
        x = x.astype(jnp.float32)

    NC = N * C
    c_blk = 64
    while NC % c_blk:
        c_blk //= 2
    grid = NC // c_blk

    x2d = x.reshape(NC * H_in, W_in)
    sel_w = _sel_w_mat(W_in, W_out).astype(x.dtype)

    out2d = pl.pallas_call(
        partial(_upsample_kernel, sf_h=sf_h),
        out_shape=jax.ShapeDtypeStruct((NC * H_out, W_out), x.dtype),
        grid=(grid,),
        in_specs=[
            pl.BlockSpec((W_in, W_out), lambda i: (0, 0)),
            pl.BlockSpec((c_blk * H_in, W_in), lambda i: (i, 0)),
        ],
        out_specs=pl.BlockSpec((c_blk * H_out, W_out), lambda i: (i, 0)),
        compiler_params=pltpu.CompilerParams(
            dimension_semantics=("parallel",),
            vmem_limit_bytes=_VMEM_LIMIT_BYTES,
        ),
    )(sel_w, x2d)

    out = out2d.reshape(N, C, H_out, W_out)
    if out.dtype != orig_dtype:
        out = out.astype(orig_dtype)
    return out


# speedup vs baseline: 2.9547x; 1.0817x over previous
import math
from functools import lru_cache, partial

import numpy as np
import jax
import jax.numpy as jnp
from jax.experimental import pallas as pl
from jax.experimental.pallas import tpu as pltpu

_VMEM_LIMIT_BYTES = 48 * 1024 * 1024


def _nearest_indices(in_dim: int, out_dim: int) -> np.ndarray:
    src = np.floor(np.arange(out_dim, dtype=np.float32) * np.float32(in_dim / out_dim))
    return np.clip(src.astype(np.int64), 0, in_dim - 1)


@lru_cache(maxsize=16)
def _sel_w_mat(w_in: int, w_out: int):
    idx = _nearest_indices(w_in, w_out)
    m = np.zeros((w_in, w_out), dtype=np.float32)
    m[idx, np.arange(w_out)] = 1.0
    return jnp.asarray(m)


def _upsample_kernel(sel_w_ref, x_ref, o_ref, *, sf_h):
    t = jnp.dot(x_ref[...], sel_w_ref[...], preferred_element_type=jnp.float32)
    r, w = t.shape
    o_ref[...] = jnp.broadcast_to(t[:, None, :], (r, sf_h, w)).reshape(r * sf_h, w)


def kernel(x):
    N, C, H_in, W_in = x.shape
    sf_h = sf_w = 2
    H_out, W_out = H_in * sf_h, W_in * sf_w

    orig_dtype = x.dtype
    if not jnp.issubdtype(x.dtype, jnp.floating):
        x = x.astype(jnp.float32)

    NC = N * C
    c_blk = 128
    while NC % c_blk:
        c_blk //= 2
    grid = NC // c_blk

    x2d = x.reshape(NC * H_in, W_in)
    sel_w = _sel_w_mat(W_in, W_out).astype(x.dtype)

    out2d = pl.pallas_call(
        partial(_upsample_kernel, sf_h=sf_h),
        out_shape=jax.ShapeDtypeStruct((NC * H_out, W_out), x.dtype),
        grid=(grid,),
        in_specs=[
            pl.BlockSpec((W_in, W_out), lambda i: (0, 0)),
            pl.BlockSpec((c_blk * H_in, W_in), lambda i: (i, 0)),
        ],
        out_specs=pl.BlockSpec((c_blk * H_out, W_out), lambda i: (i, 0)),
        compiler_params=pltpu.CompilerParams(
            dimension_semantics=("parallel",),
            vmem_limit_bytes=_VMEM_LIMIT_BYTES,
        ),
    )(sel_w, x2d)

    out = out2d.reshape(N, C, H_out, W_out)
    if out.dtype != orig_dtype:
        out = out.astype(orig_dtype)
    return out


# speedup vs baseline: 3.3508x; 1.1341x over previous
import math
from functools import lru_cache, partial

import numpy as np
import jax
import jax.numpy as jnp
from jax.experimental import pallas as pl
from jax.experimental.pallas import tpu as pltpu

_VMEM_LIMIT_BYTES = 48 * 1024 * 1024


def _nearest_indices(in_dim: int, out_dim: int) -> np.ndarray:
    src = np.floor(np.arange(out_dim, dtype=np.float32) * np.float32(in_dim / out_dim))
    return np.clip(src.astype(np.int64), 0, in_dim - 1)


@lru_cache(maxsize=16)
def _sel_w_mat(w_in: int, w_out: int):
    idx = _nearest_indices(w_in, w_out)
    m = np.zeros((w_in, w_out), dtype=np.float32)
    m[idx, np.arange(w_out)] = 1.0
    return jnp.asarray(m)


def _upsample_kernel(sel_w_ref, x_ref, o_ref, *, sf_h):
    t = jnp.dot(x_ref[...], sel_w_ref[...], preferred_element_type=jnp.float32)
    for j in range(sf_h):
        o_ref[j::sf_h, :] = t


def kernel(x):
    N, C, H_in, W_in = x.shape
    sf_h = sf_w = 2
    H_out, W_out = H_in * sf_h, W_in * sf_w

    orig_dtype = x.dtype
    if not jnp.issubdtype(x.dtype, jnp.floating):
        x = x.astype(jnp.float32)

    NC = N * C
    c_blk = 128
    while NC % c_blk:
        c_blk //= 2
    grid = NC // c_blk

    x2d = x.reshape(NC * H_in, W_in)
    sel_w = _sel_w_mat(W_in, W_out).astype(x.dtype)

    out2d = pl.pallas_call(
        partial(_upsample_kernel, sf_h=sf_h),
        out_shape=jax.ShapeDtypeStruct((NC * H_out, W_out), x.dtype),
        grid=(grid,),
        in_specs=[
            pl.BlockSpec((W_in, W_out), lambda i: (0, 0)),
            pl.BlockSpec((c_blk * H_in, W_in), lambda i: (i, 0)),
        ],
        out_specs=pl.BlockSpec((c_blk * H_out, W_out), lambda i: (i, 0)),
        compiler_params=pltpu.CompilerParams(
            dimension_semantics=("parallel",),
            vmem_limit_bytes=_VMEM_LIMIT_BYTES,
        ),
    )(sel_w, x2d)

    out = out2d.reshape(N, C, H_out, W_out)
    if out.dtype != orig_dtype:
        out = out.astype(orig_dtype)
    return out
